# baseline (speedup 1.0000x reference)
import jax
import jax.numpy as jnp
from jax import lax
from jax.experimental import pallas as pl
from jax.experimental.pallas import tpu as pltpu

_CB = 1024
_D = 256
_B = 16
_HW = 1024
_N = _B * _HW
_BETA = 0.25


def _vq_body(z_ref, w_ref, zq_ref, idx_ref, loss_ref, perp_ref, mdist_ref,
             counts_ref, acc_ref):
    b = pl.program_id(0)

    @pl.when(b == 0)
    def _init():
        counts_ref[...] = jnp.zeros_like(counts_ref)
        acc_ref[...] = jnp.zeros_like(acc_ref)

    w = w_ref[...]
    zb = z_ref[0].reshape(_D, _HW)
    zsq = jnp.sum(zb * zb, axis=0)
    wsq = jnp.sum(w * w, axis=1)
    mm = lax.dot_general(w, zb, (((1,), (0,)), ((), ())),
                         preferred_element_type=jnp.float32)
    d = (zsq[None, :] + wsq[:, None]) - 2.0 * mm
    dmin = jnp.min(d, axis=0)
    code_iota = lax.broadcasted_iota(jnp.int32, (_CB, _HW), 0)
    idx = jnp.min(jnp.where(d == dmin[None, :], code_iota, _CB), axis=0)
    oh = (code_iota == idx[None, :]).astype(jnp.float32)
    zq = lax.dot_general(w, oh, (((0,), (0,)), ((), ())),
                         preferred_element_type=jnp.float32)
    zq_ref[0] = zq.reshape(_D, 32, 32)
    idx_ref[0, 0] = idx
    counts_ref[...] += jnp.sum(oh, axis=1, keepdims=True)
    acc_ref[0, :] += jnp.broadcast_to(jnp.sum(dmin), (128,))
    acc_ref[1, :] += jnp.broadcast_to(jnp.sum(d), (128,))

    @pl.when(b == _B - 1)
    def _final():
        loss_sum = acc_ref[0, 0]
        dist_sum = acc_ref[1, 0]
        loss_ref[...] = jnp.full((8, 128), (1.0 + _BETA) * loss_sum / (_N * _D))
        mdist_ref[...] = jnp.full((8, 128), dist_sum / (_N * _CB))
        e_mean = counts_ref[...] * (1.0 / _N)
        ent = -jnp.sum(e_mean * jnp.log(e_mean + 1e-10))
        perp_ref[...] = jnp.full((8, 128), jnp.exp(ent))


def kernel(z, weight):
    out_shapes = (
        jax.ShapeDtypeStruct((_B, _D, 32, 32), jnp.float32),
        jax.ShapeDtypeStruct((_B, 1, _HW), jnp.int32),
        jax.ShapeDtypeStruct((8, 128), jnp.float32),
        jax.ShapeDtypeStruct((8, 128), jnp.float32),
        jax.ShapeDtypeStruct((8, 128), jnp.float32),
    )
    zq4, idx3, loss, perp, mdist = pl.pallas_call(
        _vq_body,
        grid=(_B,),
        in_specs=[
            pl.BlockSpec((1, _D, 32, 32), lambda b: (b, 0, 0, 0)),
            pl.BlockSpec((_CB, _D), lambda b: (0, 0)),
        ],
        out_specs=(
            pl.BlockSpec((1, _D, 32, 32), lambda b: (b, 0, 0, 0)),
            pl.BlockSpec((1, 1, _HW), lambda b: (b, 0, 0)),
            pl.BlockSpec((8, 128), lambda b: (0, 0)),
            pl.BlockSpec((8, 128), lambda b: (0, 0)),
            pl.BlockSpec((8, 128), lambda b: (0, 0)),
        ),
        out_shape=out_shapes,
        scratch_shapes=[
            pltpu.VMEM((_CB, 1), jnp.float32),
            pltpu.VMEM((2, 128), jnp.float32),
        ],
    )(z, weight)
    return (zq4, loss[0, 0], perp[0, 0],
            idx3.reshape(_N, 1), mdist[0, 0])


# speedup vs baseline: 2.3106x; 2.3106x over previous
import jax
import jax.numpy as jnp
from jax import lax
from jax.experimental import pallas as pl
from jax.experimental.pallas import tpu as pltpu

_CB = 1024
_D = 256
_B = 16
_HW = 1024
_N = _B * _HW
_BETA = 0.25


def _vq_body(z_ref, w_ref, zq_ref, idx_ref, loss_ref, perp_ref, mdist_ref,
             counts_ref, acc_ref):
    b = pl.program_id(0)

    @pl.when(b == 0)
    def _init():
        counts_ref[...] = jnp.zeros_like(counts_ref)
        acc_ref[...] = jnp.zeros_like(acc_ref)

    w = w_ref[...]
    zb = z_ref[0]
    zsq = jnp.sum(zb * zb, axis=0)
    wsq = jnp.sum(w * w, axis=1)
    mm = lax.dot_general(w, zb, (((1,), (0,)), ((), ())),
                         preferred_element_type=jnp.float32)
    d = (zsq[None, :] + wsq[:, None]) - 2.0 * mm
    dmin = jnp.min(d, axis=0)
    code_iota = lax.broadcasted_iota(jnp.int32, (_CB, _HW), 0)
    idx = jnp.min(jnp.where(d == dmin[None, :], code_iota, _CB), axis=0)
    oh = (code_iota == idx[None, :]).astype(jnp.float32)
    zq = lax.dot_general(w, oh, (((0,), (0,)), ((), ())),
                         preferred_element_type=jnp.float32)
    zq_ref[0] = zq
    idx_ref[0, 0] = idx
    counts_ref[...] += jnp.sum(oh, axis=1, keepdims=True)
    acc_ref[0, :] += jnp.broadcast_to(jnp.sum(dmin), (128,))
    acc_ref[1, :] += jnp.broadcast_to(jnp.sum(d), (128,))

    @pl.when(b == _B - 1)
    def _final():
        loss_sum = acc_ref[0, 0]
        dist_sum = acc_ref[1, 0]
        loss_ref[...] = jnp.full((8, 128), (1.0 + _BETA) * loss_sum / (_N * _D))
        mdist_ref[...] = jnp.full((8, 128), dist_sum / (_N * _CB))
        e_mean = counts_ref[...] * (1.0 / _N)
        ent = -jnp.sum(e_mean * jnp.log(e_mean + 1e-10))
        perp_ref[...] = jnp.full((8, 128), jnp.exp(ent))


def kernel(z, weight):
    z3 = z.reshape(_B, _D, _HW)
    out_shapes = (
        jax.ShapeDtypeStruct((_B, _D, _HW), jnp.float32),
        jax.ShapeDtypeStruct((_B, 1, _HW), jnp.int32),
        jax.ShapeDtypeStruct((8, 128), jnp.float32),
        jax.ShapeDtypeStruct((8, 128), jnp.float32),
        jax.ShapeDtypeStruct((8, 128), jnp.float32),
    )
    zq3, idx3, loss, perp, mdist = pl.pallas_call(
        _vq_body,
        grid=(_B,),
        in_specs=[
            pl.BlockSpec((1, _D, _HW), lambda b: (b, 0, 0)),
            pl.BlockSpec((_CB, _D), lambda b: (0, 0)),
        ],
        out_specs=(
            pl.BlockSpec((1, _D, _HW), lambda b: (b, 0, 0)),
            pl.BlockSpec((1, 1, _HW), lambda b: (b, 0, 0)),
            pl.BlockSpec((8, 128), lambda b: (0, 0)),
            pl.BlockSpec((8, 128), lambda b: (0, 0)),
            pl.BlockSpec((8, 128), lambda b: (0, 0)),
        ),
        out_shape=out_shapes,
        scratch_shapes=[
            pltpu.VMEM((_CB, 1), jnp.float32),
            pltpu.VMEM((2, 128), jnp.float32),
        ],
    )(z3, weight)
    return (zq3.reshape(_B, _D, 32, 32), loss[0, 0], perp[0, 0],
            idx3.reshape(_N, 1), mdist[0, 0])
